# Initial kernel scaffold; baseline (speedup 1.0000x reference)
import jax
import jax.numpy as jnp
from jax.experimental import pallas as pl


def kernel(input_ids, token_type_ids, input_mask, visual_embeds, visual_mask, tok_table, pos_table, seg_table, vpos_table, img_table, vseg_table, ln_g, ln_b, vln_g, vln_b):
    raise NotImplementedError("write your pallas kernel here")


# speedup vs baseline: 4.3015x; 4.3015x over previous
import functools

import jax
import jax.numpy as jnp
from jax import lax
from jax.experimental import pallas as pl
from jax.experimental.pallas import tpu as pltpu
from jax.experimental.pallas import tpu_sc as plsc

_B, _L, _F, _E, _H = 1024, 200, 32, 128, 128
_T, _P, _MAXF = 2, 512, 64
_EPS = 1e-12
_VLEN = _F + 1
_S = _L + _VLEN

_CH = 128


def _sc_gather(table, idx3):
    info = plsc.get_sparse_core_info()
    nc, ns = info.num_cores, info.num_subcores
    nw = nc * ns
    n_ch = idx3.shape[1]
    n_rows = nw * n_ch * _CH
    per_w = n_ch * _CH
    mesh = plsc.VectorSubcoreMesh(core_axis_name="c", subcore_axis_name="s")

    @functools.partial(
        pl.kernel,
        mesh=mesh,
        out_type=jax.ShapeDtypeStruct((n_rows, _E), jnp.float32),
        scratch_types=[
            pltpu.VMEM((n_ch, _CH), jnp.int32),
            pltpu.VMEM((_CH, _E), jnp.float32),
            pltpu.SemaphoreType.DMA,
        ],
    )
    def k(table_hbm, idx_hbm, out_hbm, idx_v, rows_v, sem):
        wid = lax.axis_index("s") * nc + lax.axis_index("c")
        base = wid * per_w
        pltpu.sync_copy(idx_hbm.at[wid], idx_v)

        def body(i, carry):
            pltpu.async_copy(table_hbm.at[idx_v.at[i]], rows_v, sem).wait()
            pltpu.sync_copy(rows_v, out_hbm.at[pl.ds(base + i * _CH, _CH)])
            return carry

        lax.fori_loop(0, n_ch, body, 0)

    return k(table, idx3)


def _tc_body(tok_ref, tt_ref, im_ref, ve_ref, vm_ref, pos_ref, seg_ref,
             vpos_ref, img_ref, vseg_ref, g_ref, b_ref, vg_ref, vb_ref,
             out_ref, mask_ref):
    f32 = jnp.float32
    tok = tok_ref[...]
    ttf = tt_ref[...].astype(f32)[..., None]
    seg = seg_ref[...]
    pos = pos_ref[...]
    text = tok + pos[None] + seg[0][None, None, :] + ttf * (seg[1] - seg[0])[None, None, :]

    ve = ve_ref[...]
    vmu = jnp.mean(ve, axis=-1, keepdims=True)
    vvar = jnp.mean((ve - vmu) ** 2, axis=-1, keepdims=True)
    ven = (ve - vmu) / jnp.sqrt(vvar + _EPS) * vg_ref[...] + vb_ref[...]
    vpos = vpos_ref[...]
    vseg0 = vseg_ref[...][0]
    vrows = ven + vpos[1:][None] + vseg0[None, None, :]
    vrow0 = img_ref[...][0] + vpos[0] + vseg0
    vrow0 = jnp.broadcast_to(vrow0[None, None, :], (tok.shape[0], 1, _E))

    emb = jnp.concatenate([text, vrow0, vrows], axis=1)
    mu = jnp.mean(emb, axis=-1, keepdims=True)
    var = jnp.mean((emb - mu) ** 2, axis=-1, keepdims=True)
    out_ref[...] = (emb - mu) / jnp.sqrt(var + _EPS) * g_ref[...] + b_ref[...]

    vm = vm_ref[...]
    img_mask = (jnp.sum(vm, axis=1, keepdims=True) > 0).astype(vm.dtype)
    mask_ref[...] = jnp.concatenate([im_ref[...], img_mask, vm], axis=1)


def _tc_fused(tok_rows, token_type_ids, input_mask, visual_embeds, visual_mask,
              pos_s, seg_table, vpos_s, img_table, vseg_table,
              ln_g, ln_b, vln_g, vln_b):
    bb = 8
    grid = (_B // bb,)
    const = lambda *shape: pl.BlockSpec(shape, lambda i: (0,) * len(shape))
    return pl.pallas_call(
        _tc_body,
        grid=grid,
        in_specs=[
            pl.BlockSpec((bb, _L, _E), lambda i: (i, 0, 0)),
            pl.BlockSpec((bb, _L), lambda i: (i, 0)),
            pl.BlockSpec((bb, _L), lambda i: (i, 0)),
            pl.BlockSpec((bb, _F, _E), lambda i: (i, 0, 0)),
            pl.BlockSpec((bb, _F), lambda i: (i, 0)),
            const(_L, _E),
            const(_T, _E),
            const(_VLEN, _E),
            const(1, _E),
            const(1, _E),
            const(1, _E),
            const(1, _E),
            const(1, _E),
            const(1, _E),
        ],
        out_specs=[
            pl.BlockSpec((bb, _S, _E), lambda i: (i, 0, 0)),
            pl.BlockSpec((bb, _S), lambda i: (i, 0)),
        ],
        out_shape=[
            jax.ShapeDtypeStruct((_B, _S, _E), jnp.float32),
            jax.ShapeDtypeStruct((_B, _S), jnp.int32),
        ],
    )(tok_rows, token_type_ids, input_mask, visual_embeds, visual_mask,
      pos_s, seg_table, vpos_s, img_table, vseg_table,
      ln_g.reshape(1, _E), ln_b.reshape(1, _E),
      vln_g.reshape(1, _E), vln_b.reshape(1, _E))


def kernel(input_ids, token_type_ids, input_mask, visual_embeds, visual_mask,
           tok_table, pos_table, seg_table, vpos_table, img_table, vseg_table,
           ln_g, ln_b, vln_g, vln_b):
    info = plsc.get_sparse_core_info()
    nw = info.num_cores * info.num_subcores
    n_ch = (_B * _L) // (nw * _CH)
    idx3 = input_ids.reshape(nw, n_ch, _CH)
    tok_rows = _sc_gather(tok_table, idx3).reshape(_B, _L, _E)
    emb, mask = _tc_fused(
        tok_rows, token_type_ids, input_mask, visual_embeds, visual_mask,
        pos_table[:_L], seg_table, vpos_table[:_VLEN], img_table, vseg_table,
        ln_g, ln_b, vln_g, vln_b)
    return emb, mask


# speedup vs baseline: 4.4011x; 1.0231x over previous
import functools

import jax
import jax.numpy as jnp
from jax import lax
from jax.experimental import pallas as pl
from jax.experimental.pallas import tpu as pltpu
from jax.experimental.pallas import tpu_sc as plsc

_B, _L, _F, _E, _H = 1024, 200, 32, 128, 128
_T, _P, _MAXF = 2, 512, 64
_EPS = 1e-12
_VLEN = _F + 1
_S = _L + _VLEN

_CH = 128


def _sc_gather(table, idx3):
    info = plsc.get_sparse_core_info()
    nc, ns = info.num_cores, info.num_subcores
    nw = nc * ns
    n_ch = idx3.shape[1]
    n_rows = nw * n_ch * _CH
    per_w = n_ch * _CH
    mesh = plsc.VectorSubcoreMesh(core_axis_name="c", subcore_axis_name="s")

    @functools.partial(
        pl.kernel,
        mesh=mesh,
        out_type=jax.ShapeDtypeStruct((n_rows, _E), jnp.float32),
        scratch_types=[
            pltpu.VMEM((n_ch, _CH), jnp.int32),
            pltpu.VMEM((_CH, _E), jnp.float32),
            pltpu.SemaphoreType.DMA,
        ],
    )
    def k(table_hbm, idx_hbm, out_hbm, idx_v, rows_v, sem):
        wid = lax.axis_index("s") * nc + lax.axis_index("c")
        base = wid * per_w
        pltpu.sync_copy(idx_hbm.at[wid], idx_v)

        def body(i, carry):
            pltpu.async_copy(table_hbm.at[idx_v.at[i]], rows_v, sem).wait()
            pltpu.sync_copy(rows_v, out_hbm.at[pl.ds(base + i * _CH, _CH)])
            return carry

        lax.fori_loop(0, n_ch, body, 0)

    return k(table, idx3)


def _tc_body(tok_ref, tt_ref, im_ref, ve_ref, vm_ref, pos_ref, seg_ref,
             vpos_ref, img_ref, vseg_ref, g_ref, b_ref, vg_ref, vb_ref,
             out_ref, mask_ref):
    f32 = jnp.float32
    tok = tok_ref[...]
    ttf = tt_ref[...].astype(f32)[..., None]
    seg = seg_ref[...]
    pos = pos_ref[...]
    text = tok + pos[None] + seg[0][None, None, :] + ttf * (seg[1] - seg[0])[None, None, :]

    ve = ve_ref[...]
    vmu = jnp.mean(ve, axis=-1, keepdims=True)
    vvar = jnp.mean(ve * ve, axis=-1, keepdims=True) - vmu * vmu
    ven = (ve - vmu) * lax.rsqrt(vvar + _EPS) * vg_ref[...] + vb_ref[...]
    vpos = vpos_ref[...]
    vseg0 = vseg_ref[...][0]
    vrows = ven + vpos[1:][None] + vseg0[None, None, :]
    vrow0 = img_ref[...][0] + vpos[0] + vseg0
    vrow0 = jnp.broadcast_to(vrow0[None, None, :], (tok.shape[0], 1, _E))

    emb = jnp.concatenate([text, vrow0, vrows], axis=1)
    mu = jnp.mean(emb, axis=-1, keepdims=True)
    var = jnp.mean(emb * emb, axis=-1, keepdims=True) - mu * mu
    out_ref[...] = (emb - mu) * lax.rsqrt(var + _EPS) * g_ref[...] + b_ref[...]

    vm = vm_ref[...]
    img_mask = (jnp.sum(vm, axis=1, keepdims=True) > 0).astype(vm.dtype)
    mask_ref[...] = jnp.concatenate([im_ref[...], img_mask, vm], axis=1)


def _tc_fused(tok_rows, token_type_ids, input_mask, visual_embeds, visual_mask,
              pos_s, seg_table, vpos_s, img_table, vseg_table,
              ln_g, ln_b, vln_g, vln_b):
    bb = 8
    grid = (_B // bb,)
    const = lambda *shape: pl.BlockSpec(shape, lambda i: (0,) * len(shape))
    return pl.pallas_call(
        _tc_body,
        grid=grid,
        in_specs=[
            pl.BlockSpec((bb, _L, _E), lambda i: (i, 0, 0)),
            pl.BlockSpec((bb, _L), lambda i: (i, 0)),
            pl.BlockSpec((bb, _L), lambda i: (i, 0)),
            pl.BlockSpec((bb, _F, _E), lambda i: (i, 0, 0)),
            pl.BlockSpec((bb, _F), lambda i: (i, 0)),
            const(_L, _E),
            const(_T, _E),
            const(_VLEN, _E),
            const(1, _E),
            const(1, _E),
            const(1, _E),
            const(1, _E),
            const(1, _E),
            const(1, _E),
        ],
        out_specs=[
            pl.BlockSpec((bb, _S, _E), lambda i: (i, 0, 0)),
            pl.BlockSpec((bb, _S), lambda i: (i, 0)),
        ],
        out_shape=[
            jax.ShapeDtypeStruct((_B, _S, _E), jnp.float32),
            jax.ShapeDtypeStruct((_B, _S), jnp.int32),
        ],
    )(tok_rows, token_type_ids, input_mask, visual_embeds, visual_mask,
      pos_s, seg_table, vpos_s, img_table, vseg_table,
      ln_g.reshape(1, _E), ln_b.reshape(1, _E),
      vln_g.reshape(1, _E), vln_b.reshape(1, _E))


def kernel(input_ids, token_type_ids, input_mask, visual_embeds, visual_mask,
           tok_table, pos_table, seg_table, vpos_table, img_table, vseg_table,
           ln_g, ln_b, vln_g, vln_b):
    info = plsc.get_sparse_core_info()
    nw = info.num_cores * info.num_subcores
    n_ch = (_B * _L) // (nw * _CH)
    idx3 = input_ids.reshape(nw, n_ch, _CH)
    tok_rows = _sc_gather(tok_table, idx3).reshape(_B, _L, _E)
    emb, mask = _tc_fused(
        tok_rows, token_type_ids, input_mask, visual_embeds, visual_mask,
        pos_table[:_L], seg_table, vpos_table[:_VLEN], img_table, vseg_table,
        ln_g, ln_b, vln_g, vln_b)
    return emb, mask
